# baseline (speedup 1.0000x reference)
import jax
import jax.numpy as jnp
import numpy as np
from jax.experimental import pallas as pl
from jax.experimental.pallas import tpu as pltpu

_PS = 32
_NB = 36
_CH = 16
_BB = 128

_ATAN_C = (
    0.9999999975460191, -0.3333328229551171, 0.19998230640374667,
    -0.14261573680269654, 0.10940198965065628, -0.08372063947871689,
    0.05746355784645889, -0.030717508903292983, 0.010680719445998774,
    -0.001743701143875627,
)


def _gauss_row():
    half = _PS / 2.0
    sigma2 = 0.9 * half * half
    x = np.linspace(-half, half, _PS)
    xv, yv = np.meshgrid(x, x, indexing="xy")
    k = np.exp(-(xv ** 2 + yv ** 2) / sigma2)
    k = k / np.sum(k)
    return (10.0 * k).reshape(1, _PS * _PS).astype(np.float32)


def _atan2(y, x):
    ax = jnp.abs(x)
    ay = jnp.abs(y)
    hi = jnp.maximum(jnp.maximum(ax, ay), np.float32(1e-30))
    lo = jnp.minimum(ax, ay)
    r = 1.0 / hi
    z = lo * r
    z = z + (lo - z * hi) * r
    u = z * z
    p = jnp.float32(_ATAN_C[-1])
    for c in _ATAN_C[-2::-1]:
        p = p * u + np.float32(c)
    a = z * p
    a = jnp.where(ay > ax, np.float32(np.pi / 2) - a, a)
    a = jnp.where(x < 0, np.float32(np.pi) - a, a)
    return jnp.where(y < 0, -a, a)


def _chunk(x, gk, mc0, mc31, mr0, mr31, o_rows):
    n = _PS * _PS
    xl = jnp.where(mc0, x, pltpu.roll(x, 1, axis=1))
    xr = jnp.where(mc31, x, pltpu.roll(x, n - 1, axis=1))
    gx = 0.5 * (xl - xr)
    xu = jnp.where(mr0, x, pltpu.roll(x, _PS, axis=1))
    xd = jnp.where(mr31, x, pltpu.roll(x, n - _PS, axis=1))
    gy = 0.5 * (xu - xd)

    mag = jnp.sqrt(gx * gx + gy * gy + np.float32(1e-10)) * gk
    ori = _atan2(gy, gx)

    o = (np.float32(_NB) * (ori + np.float32(np.pi))) / np.float32(2.0 * np.pi)
    bo0 = jnp.floor(o)
    w = ((bo0 + 1.0) - o) * mag
    bo = jnp.where(bo0 >= _NB, np.float32(0.0), bo0)

    cols = []
    for k in range(_NB):
        m = bo == np.float32(k)
        cols.append(jnp.sum(jnp.where(m, w, 0.0), axis=1, keepdims=True))
    hist = jnp.concatenate(cols, axis=1) * np.float32(1.0 / n)

    z = jnp.zeros((_CH, 1), jnp.float32)
    hl = jnp.concatenate([z, hist[:, :-1]], axis=1)
    hr = jnp.concatenate([hist[:, 1:], z], axis=1)
    sm = 0.33 * hl + 0.34 * hist + 0.33 * hr

    mx = jnp.max(sm, axis=1, keepdims=True)
    io = jax.lax.broadcasted_iota(jnp.int32, sm.shape, 1).astype(jnp.float32)
    idx = jnp.min(jnp.where(sm == mx, io, np.float32(_NB)), axis=1,
                  keepdims=True)
    o_rows[...] = -(np.float32(2.0 * np.pi / _NB) * idx - np.float32(np.pi))


def _body(x_ref, gk_ref, o_ref):
    n = _PS * _PS
    gk = gk_ref[...]
    lane = jax.lax.broadcasted_iota(jnp.int32, (1, n), 1)
    col = lane % _PS
    row = lane // _PS
    mc0 = col == 0
    mc31 = col == _PS - 1
    mr0 = row == 0
    mr31 = row == _PS - 1

    for c in range(_BB // _CH):
        xc = x_ref[c * _CH:(c + 1) * _CH, :]
        _chunk(xc, gk, mc0, mc31, mr0, mr31,
               o_ref.at[c * _CH:(c + 1) * _CH, :])


@jax.jit
def kernel(x):
    b = x.shape[0]
    x2 = x.reshape(b, _PS * _PS)
    grid = (b // _BB,)
    out = pl.pallas_call(
        _body,
        grid=grid,
        in_specs=[
            pl.BlockSpec((_BB, _PS * _PS), lambda i: (i, 0)),
            pl.BlockSpec((1, _PS * _PS), lambda i: (0, 0)),
        ],
        out_specs=pl.BlockSpec((_BB, 1), lambda i: (i, 0)),
        out_shape=jax.ShapeDtypeStruct((b, 1), jnp.float32),
        compiler_params=pltpu.CompilerParams(
            dimension_semantics=("parallel",),
        ),
    )(x2, jnp.asarray(_gauss_row()))
    return out.reshape(b)


# speedup vs baseline: 1.3036x; 1.3036x over previous
import jax
import jax.numpy as jnp
import numpy as np
from jax.experimental import pallas as pl
from jax.experimental.pallas import tpu as pltpu

_PS = 32
_NB = 36
_BB = 256


def _gauss_row():
    half = _PS / 2.0
    sigma2 = 0.9 * half * half
    x = np.linspace(-half, half, _PS)
    xv, yv = np.meshgrid(x, x, indexing="xy")
    k = np.exp(-(xv ** 2 + yv ** 2) / sigma2)
    k = k / np.sum(k)
    return (10.0 * k).reshape(1, _PS * _PS).astype(np.float32)


def _body(x_ref, gk_ref, o_ref):
    x = x_ref[...]
    n = _PS * _PS

    lane = jax.lax.broadcasted_iota(jnp.int32, (1, n), 1)
    col = lane % _PS
    row = lane // _PS

    xl = jnp.where(col == 0, x, pltpu.roll(x, 1, axis=1))
    xr = jnp.where(col == _PS - 1, x, pltpu.roll(x, n - 1, axis=1))
    gx = 0.5 * (xl - xr)

    xu = jnp.where(row == 0, x, pltpu.roll(x, _PS, axis=1))
    xd = jnp.where(row == _PS - 1, x, pltpu.roll(x, n - _PS, axis=1))
    gy = 0.5 * (xu - xd)

    gk = gk_ref[...]
    mag = jnp.sqrt(gx * gx + gy * gy + 1e-10) * gk
    ori = jnp.arctan2(gy, gx)

    o_big = _NB * (ori + np.float32(np.pi)) / np.float32(2.0 * np.pi)
    bo0 = jnp.floor(o_big)
    w = ((bo0 + 1.0) - o_big) * mag
    bo = jnp.where(bo0 >= _NB, np.float32(0.0), bo0)

    inv_n = np.float32(1.0 / n)
    cols = []
    for k in range(_NB):
        m = bo == np.float32(k)
        cols.append(jnp.sum(jnp.where(m, w, 0.0), axis=1, keepdims=True))
    hist = jnp.concatenate(cols, axis=1) * inv_n

    z = jnp.zeros((hist.shape[0], 1), jnp.float32)
    hl = jnp.concatenate([z, hist[:, :-1]], axis=1)
    hr = jnp.concatenate([hist[:, 1:], z], axis=1)
    sm = 0.33 * hl + 0.34 * hist + 0.33 * hr

    mx = jnp.max(sm, axis=1, keepdims=True)
    io = jax.lax.broadcasted_iota(jnp.int32, sm.shape, 1).astype(jnp.float32)
    idx = jnp.min(jnp.where(sm == mx, io, np.float32(_NB)), axis=1,
                  keepdims=True)
    ang = -(np.float32(2.0 * np.pi / _NB) * idx - np.float32(np.pi))
    o_ref[...] = jnp.broadcast_to(ang, (ang.shape[0], 128))


@jax.jit
def kernel(x):
    b = x.shape[0]
    x2 = x.reshape(b, _PS * _PS)
    grid = (b // _BB,)
    out = pl.pallas_call(
        _body,
        grid=grid,
        in_specs=[
            pl.BlockSpec((_BB, _PS * _PS), lambda i: (i, 0)),
            pl.BlockSpec((1, _PS * _PS), lambda i: (0, 0)),
        ],
        out_specs=pl.BlockSpec((_BB, 128), lambda i: (i, 0)),
        out_shape=jax.ShapeDtypeStruct((b, 128), jnp.float32),
        compiler_params=pltpu.CompilerParams(
            dimension_semantics=("parallel",),
        ),
    )(x2, jnp.asarray(_gauss_row()))
    return out[:, 0]


# speedup vs baseline: 1.3069x; 1.0025x over previous
import jax
import jax.numpy as jnp
import numpy as np
from jax.experimental import pallas as pl
from jax.experimental.pallas import tpu as pltpu

_PS = 32
_NB = 36
_BB = 512


def _gauss_row():
    half = _PS / 2.0
    sigma2 = 0.9 * half * half
    x = np.linspace(-half, half, _PS)
    xv, yv = np.meshgrid(x, x, indexing="xy")
    k = np.exp(-(xv ** 2 + yv ** 2) / sigma2)
    k = k / np.sum(k)
    return (10.0 * k).reshape(1, _PS * _PS).astype(np.float32)


def _body(x_ref, gk_ref, o_ref):
    x = x_ref[...]
    n = _PS * _PS

    lane = jax.lax.broadcasted_iota(jnp.int32, (1, n), 1)
    col = lane % _PS
    row = lane // _PS

    xl = jnp.where(col == 0, x, pltpu.roll(x, 1, axis=1))
    xr = jnp.where(col == _PS - 1, x, pltpu.roll(x, n - 1, axis=1))
    gx = 0.5 * (xl - xr)

    xu = jnp.where(row == 0, x, pltpu.roll(x, _PS, axis=1))
    xd = jnp.where(row == _PS - 1, x, pltpu.roll(x, n - _PS, axis=1))
    gy = 0.5 * (xu - xd)

    gk = gk_ref[...]
    mag = jnp.sqrt(gx * gx + gy * gy + 1e-10) * gk
    ori = jnp.arctan2(gy, gx)

    o_big = _NB * (ori + np.float32(np.pi)) / np.float32(2.0 * np.pi)
    bo0 = jnp.floor(o_big)
    w = ((bo0 + 1.0) - o_big) * mag
    bo = jnp.where(bo0 >= _NB, np.float32(0.0), bo0)

    inv_n = np.float32(1.0 / n)
    cols = []
    for k in range(_NB):
        m = bo == np.float32(k)
        cols.append(jnp.sum(jnp.where(m, w, 0.0), axis=1, keepdims=True))
    hist = jnp.concatenate(cols, axis=1) * inv_n

    z = jnp.zeros((hist.shape[0], 1), jnp.float32)
    hl = jnp.concatenate([z, hist[:, :-1]], axis=1)
    hr = jnp.concatenate([hist[:, 1:], z], axis=1)
    sm = 0.33 * hl + 0.34 * hist + 0.33 * hr

    mx = jnp.max(sm, axis=1, keepdims=True)
    io = jax.lax.broadcasted_iota(jnp.int32, sm.shape, 1).astype(jnp.float32)
    idx = jnp.min(jnp.where(sm == mx, io, np.float32(_NB)), axis=1,
                  keepdims=True)
    ang = -(np.float32(2.0 * np.pi / _NB) * idx - np.float32(np.pi))
    o_ref[...] = jnp.broadcast_to(ang, (ang.shape[0], 128))


@jax.jit
def kernel(x):
    b = x.shape[0]
    x2 = x.reshape(b, _PS * _PS)
    grid = (b // _BB,)
    out = pl.pallas_call(
        _body,
        grid=grid,
        in_specs=[
            pl.BlockSpec((_BB, _PS * _PS), lambda i: (i, 0)),
            pl.BlockSpec((1, _PS * _PS), lambda i: (0, 0)),
        ],
        out_specs=pl.BlockSpec((_BB, 128), lambda i: (i, 0)),
        out_shape=jax.ShapeDtypeStruct((b, 128), jnp.float32),
        compiler_params=pltpu.CompilerParams(
            dimension_semantics=("parallel",),
        ),
    )(x2, jnp.asarray(_gauss_row()))
    return out[:, 0]
